# baseline (speedup 1.0000x reference)
import functools

import jax
import jax.numpy as jnp
from jax import lax
from jax.experimental import pallas as pl
from jax.experimental.pallas import tpu as pltpu
from jax.experimental.pallas import tpu_sc as plsc

N = 10000
D_IN = 128
DH = 32
DO = 16
NC = 2
NS = 16
NW = NC * NS
EPW = 320000 // NW
CW = 125
CH = EPW // CW
NPAD = 10240
RPT = NPAD // NS
ZB = 128
NB = 8


def _mesh():
    return plsc.VectorSubcoreMesh(
        core_axis_name="c", subcore_axis_name="s", num_cores=NC, num_subcores=NS
    )


_SC_PARAMS = pltpu.CompilerParams(use_tc_tiling_on_sc=False)


def _sc_degree(dst_r):
    @functools.partial(
        pl.kernel,
        out_type=jax.ShapeDtypeStruct((NC, NPAD, 16), jnp.float32),
        mesh=_mesh(),
        compiler_params=_SC_PARAMS,
        scratch_types=[
            pltpu.VMEM((CH, CW), jnp.int32),
            pltpu.VMEM((ZB, 16), jnp.float32),
            pltpu.VMEM((CW, 16), jnp.float32),
            pltpu.VMEM_SHARED((NPAD, 16), jnp.float32),
            pltpu.SemaphoreType.DMA,
        ],
    )
    def k(dst_hbm, out_hbm, idx_v, zbuf_v, ones_v, acc_sh, sem):
        c = lax.axis_index("c")
        s = lax.axis_index("s")
        wid = c * NS + s
        pltpu.sync_copy(dst_hbm.at[wid], idx_v)

        def zrow(i, carry):
            zbuf_v[i, :] = jnp.zeros((16,), jnp.float32)
            return carry

        lax.fori_loop(0, ZB, zrow, 0)

        def orow(i, carry):
            ones_v[i, :] = jnp.full((16,), 1.0, jnp.float32)
            return carry

        lax.fori_loop(0, CW, orow, 0)
        base = s * RPT
        for r in range(RPT // ZB):
            pltpu.sync_copy(zbuf_v, acc_sh.at[pl.ds(base + r * ZB, ZB)])
        plsc.subcore_barrier()

        def chunk(j, carry):
            pltpu.async_copy(ones_v, acc_sh.at[idx_v.at[j]], sem, add=True)
            return carry

        lax.fori_loop(0, CH, chunk, 0)

        def drain(j, carry):
            pltpu.make_async_copy(ones_v, acc_sh.at[idx_v.at[j]], sem).wait()
            return carry

        lax.fori_loop(0, CH, drain, 0)
        plsc.subcore_barrier()
        pltpu.sync_copy(acc_sh.at[pl.ds(base, RPT)], out_hbm.at[c, pl.ds(base, RPT)])

    return k(dst_r)


def _sc_propagate(hp, src_r, dst_r):
    @functools.partial(
        pl.kernel,
        out_type=jax.ShapeDtypeStruct((NC, NPAD, DH), jnp.float32),
        mesh=_mesh(),
        compiler_params=_SC_PARAMS,
        scratch_types=[
            pltpu.VMEM((CH, CW), jnp.int32),
            pltpu.VMEM((CH, CW), jnp.int32),
            pltpu.VMEM((ZB, DH), jnp.float32),
            pltpu.VMEM((NB, CW, DH), jnp.float32),
            pltpu.VMEM_SHARED((NPAD, DH), jnp.float32),
            pltpu.SemaphoreType.DMA((NB,)),
            pltpu.SemaphoreType.DMA((NB,)),
        ],
    )
    def k(hp_hbm, src_hbm, dst_hbm, out_hbm, sidx_v, didx_v, zbuf_v, rows_v,
          acc_sh, gsem, ssem):
        c = lax.axis_index("c")
        s = lax.axis_index("s")
        wid = c * NS + s
        pltpu.sync_copy(src_hbm.at[wid], sidx_v)
        pltpu.sync_copy(dst_hbm.at[wid], didx_v)

        def zrow(i, carry):
            zbuf_v[i, pl.ds(0, 16)] = jnp.zeros((16,), jnp.float32)
            zbuf_v[i, pl.ds(16, 16)] = jnp.zeros((16,), jnp.float32)
            return carry

        lax.fori_loop(0, ZB, zrow, 0)
        base = s * RPT
        for r in range(RPT // ZB):
            pltpu.sync_copy(zbuf_v, acc_sh.at[pl.ds(base + r * ZB, ZB)])
        plsc.subcore_barrier()

        def gather(i, b):
            pltpu.async_copy(hp_hbm.at[sidx_v.at[i]], rows_v.at[b], gsem.at[b])

        def gather_wait(i, b):
            pltpu.make_async_copy(
                hp_hbm.at[sidx_v.at[i]], rows_v.at[b], gsem.at[b]
            ).wait()

        def scatter(i, b):
            pltpu.async_copy(
                rows_v.at[b], acc_sh.at[didx_v.at[i]], ssem.at[b], add=True
            )

        def scatter_wait(i, b):
            pltpu.make_async_copy(
                rows_v.at[b], acc_sh.at[didx_v.at[i]], ssem.at[b]
            ).wait()

        def step(i, p, wait_prev_scatter, prefetch):
            gather_wait(i, p)
            scatter(i, p)
            if prefetch:
                bp = (p + NB - 1) % NB
                if wait_prev_scatter:
                    scatter_wait(i - 1, bp)
                gather(i + NB - 1, bp)

        for b in range(NB - 1):
            gather(b, b)
        step(0, 0, False, True)
        for p in range(1, NB):
            step(p, p, True, True)

        def group(g, carry):
            for p in range(NB):
                step(g * NB + p, p, True, True)
            return carry

        lax.fori_loop(1, CH // NB - 1, group, 0)
        i0 = CH - NB
        step(i0, 0, True, True)
        for p in range(1, NB):
            step(i0 + p, p, False, False)
        for b in range(NB):
            scatter_wait(i0 + b, b)
        plsc.subcore_barrier()
        pltpu.sync_copy(acc_sh.at[pl.ds(base, RPT)], out_hbm.at[c, pl.ds(base, RPT)])

    return k(hp, src_r, dst_r)


_R = 2560
_G = NPAD // _R
_RD = _R * 16 // 128
_RH = _R * DH // 128


def _unpack32(p):
    parts = [p[:, DH * g:DH * (g + 1)].reshape(-1, 1, DH) for g in range(4)]
    return jnp.concatenate(parts, axis=1).reshape(-1, DH)


def _pack32(h):
    h4 = h.reshape(-1, 4, DH)
    return jnp.concatenate([h4[:, g, :] for g in range(4)], axis=1)


def _dis_of(d_ref):
    d = d_ref[0] + d_ref[1]
    cols = [d[:, 16 * g:16 * g + 1].reshape(-1, 1, 1) for g in range(8)]
    deg = jnp.concatenate(cols, axis=1).reshape(_R, 1)
    return lax.rsqrt(deg + 1.0)


def _tc_first(x, W1, degp):
    def body(x_ref, w_ref, d_ref, o_ref):
        dis = _dis_of(d_ref)
        h = jnp.dot(x_ref[...], w_ref[...], preferred_element_type=jnp.float32)
        o_ref[...] = _pack32(dis * h)

    return pl.pallas_call(
        body,
        grid=(_G,),
        in_specs=[
            pl.BlockSpec((_R, D_IN), lambda i: (i, 0)),
            pl.BlockSpec((D_IN, DH), lambda i: (0, 0)),
            pl.BlockSpec((2, _RD, 128), lambda i: (0, i, 0)),
        ],
        out_specs=pl.BlockSpec((_RH, 128), lambda i: (i, 0)),
        out_shape=jax.ShapeDtypeStruct((NPAD * DH // 128, 128), jnp.float32),
    )(x, W1, degp)


def _tc_mid(degp, sp, hp, W2, b1):
    def body(d_ref, sp_ref, hp_ref, w_ref, b_ref, o_ref):
        dis = _dis_of(d_ref)
        seg = _unpack32(sp_ref[0] + sp_ref[1] + hp_ref[...])
        h1 = jnp.maximum(dis * seg + b_ref[...], 0.0)
        h2 = dis * jnp.dot(h1, w_ref[...], preferred_element_type=jnp.float32)
        o_ref[...] = _pack32(h2)

    return pl.pallas_call(
        body,
        grid=(_G,),
        in_specs=[
            pl.BlockSpec((2, _RD, 128), lambda i: (0, i, 0)),
            pl.BlockSpec((2, _RH, 128), lambda i: (0, i, 0)),
            pl.BlockSpec((_RH, 128), lambda i: (i, 0)),
            pl.BlockSpec((DH, DH), lambda i: (0, 0)),
            pl.BlockSpec((1, DH), lambda i: (0, 0)),
        ],
        out_specs=pl.BlockSpec((_RH, 128), lambda i: (i, 0)),
        out_shape=jax.ShapeDtypeStruct((NPAD * DH // 128, 128), jnp.float32),
    )(degp, sp, hp, W2, b1)


def _tc_last(degp, sp, hp, b2, Wl, bl):
    def body(d_ref, sp_ref, hp_ref, b2_ref, wl_ref, bl_ref, o_ref):
        dis = _dis_of(d_ref)
        seg = _unpack32(sp_ref[0] + sp_ref[1] + hp_ref[...])
        h2 = jnp.maximum(dis * seg + b2_ref[...], 0.0)
        o = jnp.dot(h2, wl_ref[...], preferred_element_type=jnp.float32) + bl_ref[...]
        m = jnp.max(o, axis=1, keepdims=True)
        lse = jnp.log(jnp.sum(jnp.exp(o - m), axis=1, keepdims=True)) + m
        o_ref[...] = o - lse

    return pl.pallas_call(
        body,
        grid=(_G,),
        in_specs=[
            pl.BlockSpec((2, _RD, 128), lambda i: (0, i, 0)),
            pl.BlockSpec((2, _RH, 128), lambda i: (0, i, 0)),
            pl.BlockSpec((_RH, 128), lambda i: (i, 0)),
            pl.BlockSpec((1, DH), lambda i: (0, 0)),
            pl.BlockSpec((DH, DO), lambda i: (0, 0)),
            pl.BlockSpec((1, DO), lambda i: (0, 0)),
        ],
        out_specs=pl.BlockSpec((_R, DO), lambda i: (i, 0)),
        out_shape=jax.ShapeDtypeStruct((N, DO), jnp.float32),
    )(degp, sp, hp, b2, Wl, bl)


def kernel(x, edge_index, W1, b1, W2, b2, Wl, bl):
    ei = edge_index.astype(jnp.int32)
    src_r = ei[0].reshape(NW, CH, CW)
    dst_r = ei[1].reshape(NW, CH, CW)

    degp = _sc_degree(dst_r).reshape(2, NPAD * 16 // 128, 128)
    h1p = _tc_first(x, W1, degp)
    s1p = _sc_propagate(h1p.reshape(NPAD, DH), src_r, dst_r)
    h2p = _tc_mid(degp, s1p.reshape(2, NPAD * DH // 128, 128), h1p,
                  W2, b1.reshape(1, DH))
    s2p = _sc_propagate(h2p.reshape(NPAD, DH), src_r, dst_r)
    return _tc_last(degp, s2p.reshape(2, NPAD * DH // 128, 128), h2p,
                    b2.reshape(1, DH), Wl, bl.reshape(1, DO))


# speedup vs baseline: 1.2804x; 1.2804x over previous
import functools

import jax
import jax.numpy as jnp
from jax import lax
from jax.experimental import pallas as pl
from jax.experimental.pallas import tpu as pltpu
from jax.experimental.pallas import tpu_sc as plsc

N = 10000
D_IN = 128
DH = 32
DO = 16
NC = 2
NS = 16
NW = NC * NS
EPW = 320000 // NW
CW = 125
CH = EPW // CW
NPAD = 10240
RPT = NPAD // NS
ZB = 128
NB = 8


def _mesh():
    return plsc.VectorSubcoreMesh(
        core_axis_name="c", subcore_axis_name="s", num_cores=NC, num_subcores=NS
    )


_SC_PARAMS = pltpu.CompilerParams(use_tc_tiling_on_sc=False)


def _sc_degree(dst_r):
    @functools.partial(
        pl.kernel,
        out_type=jax.ShapeDtypeStruct((NC, NPAD, DH), jnp.float32),
        mesh=_mesh(),
        compiler_params=_SC_PARAMS,
        scratch_types=[
            pltpu.VMEM((CH, CW), jnp.int32),
            pltpu.VMEM((ZB, DH), jnp.float32),
            pltpu.VMEM((CW, DH), jnp.float32),
            pltpu.VMEM_SHARED((NPAD, DH), jnp.float32),
            pltpu.SemaphoreType.DMA,
        ],
    )
    def k(dst_hbm, out_hbm, idx_v, zbuf_v, ones_v, acc_sh, sem):
        c = lax.axis_index("c")
        s = lax.axis_index("s")
        wid = c * NS + s
        pltpu.sync_copy(dst_hbm.at[wid], idx_v)

        def zrow(i, carry):
            zbuf_v[i, pl.ds(0, 16)] = jnp.zeros((16,), jnp.float32)
            zbuf_v[i, pl.ds(16, 16)] = jnp.zeros((16,), jnp.float32)
            return carry

        lax.fori_loop(0, ZB, zrow, 0)

        def orow(i, carry):
            ones_v[i, pl.ds(0, 16)] = jnp.full((16,), 1.0, jnp.float32)
            ones_v[i, pl.ds(16, 16)] = jnp.full((16,), 1.0, jnp.float32)
            return carry

        lax.fori_loop(0, CW, orow, 0)
        base = s * RPT
        for r in range(RPT // ZB):
            pltpu.sync_copy(zbuf_v, acc_sh.at[pl.ds(base + r * ZB, ZB)])
        plsc.subcore_barrier()

        def chunk(j, carry):
            pltpu.async_copy(ones_v, acc_sh.at[idx_v.at[j]], sem, add=True)
            return carry

        lax.fori_loop(0, CH, chunk, 0)

        def drain(j, carry):
            pltpu.make_async_copy(ones_v, acc_sh.at[idx_v.at[j]], sem).wait()
            return carry

        lax.fori_loop(0, CH, drain, 0)
        plsc.subcore_barrier()
        pltpu.sync_copy(acc_sh.at[pl.ds(base, RPT)], out_hbm.at[c, pl.ds(base, RPT)])

    return k(dst_r)


def _sc_propagate(hp, src_r, dst_r):
    @functools.partial(
        pl.kernel,
        out_type=jax.ShapeDtypeStruct((NC, NPAD, DH), jnp.float32),
        mesh=_mesh(),
        compiler_params=_SC_PARAMS,
        scratch_types=[
            pltpu.VMEM((CH, CW), jnp.int32),
            pltpu.VMEM((CH, CW), jnp.int32),
            pltpu.VMEM((ZB, DH), jnp.float32),
            pltpu.VMEM((NB, CW, DH), jnp.float32),
            pltpu.VMEM_SHARED((NPAD, DH), jnp.float32),
            pltpu.SemaphoreType.DMA((NB,)),
            pltpu.SemaphoreType.DMA((NB,)),
        ],
    )
    def k(hp_hbm, src_hbm, dst_hbm, out_hbm, sidx_v, didx_v, zbuf_v, rows_v,
          acc_sh, gsem, ssem):
        c = lax.axis_index("c")
        s = lax.axis_index("s")
        wid = c * NS + s
        pltpu.sync_copy(src_hbm.at[wid], sidx_v)
        pltpu.sync_copy(dst_hbm.at[wid], didx_v)

        def zrow(i, carry):
            zbuf_v[i, pl.ds(0, 16)] = jnp.zeros((16,), jnp.float32)
            zbuf_v[i, pl.ds(16, 16)] = jnp.zeros((16,), jnp.float32)
            return carry

        lax.fori_loop(0, ZB, zrow, 0)
        base = s * RPT
        for r in range(RPT // ZB):
            pltpu.sync_copy(zbuf_v, acc_sh.at[pl.ds(base + r * ZB, ZB)])
        plsc.subcore_barrier()

        def gather(i, b):
            pltpu.async_copy(hp_hbm.at[sidx_v.at[i]], rows_v.at[b], gsem.at[b])

        def gather_wait(i, b):
            pltpu.make_async_copy(
                hp_hbm.at[sidx_v.at[i]], rows_v.at[b], gsem.at[b]
            ).wait()

        def scatter(i, b):
            pltpu.async_copy(
                rows_v.at[b], acc_sh.at[didx_v.at[i]], ssem.at[b], add=True
            )

        def scatter_wait(i, b):
            pltpu.make_async_copy(
                rows_v.at[b], acc_sh.at[didx_v.at[i]], ssem.at[b]
            ).wait()

        def step(i, p, wait_prev_scatter, prefetch):
            gather_wait(i, p)
            scatter(i, p)
            if prefetch:
                bp = (p + NB - 1) % NB
                if wait_prev_scatter:
                    scatter_wait(i - 1, bp)
                gather(i + NB - 1, bp)

        for b in range(NB - 1):
            gather(b, b)
        step(0, 0, False, True)
        for p in range(1, NB):
            step(p, p, True, True)

        def group(g, carry):
            for p in range(NB):
                step(g * NB + p, p, True, True)
            return carry

        lax.fori_loop(1, CH // NB - 1, group, 0)
        i0 = CH - NB
        step(i0, 0, True, True)
        for p in range(1, NB):
            step(i0 + p, p, False, False)
        for b in range(NB):
            scatter_wait(i0 + b, b)
        plsc.subcore_barrier()
        pltpu.sync_copy(acc_sh.at[pl.ds(base, RPT)], out_hbm.at[c, pl.ds(base, RPT)])

    return k(hp, src_r, dst_r)


_R = 2560
_G = NPAD // _R
_RD = _R * 16 // 128
_RH = _R * DH // 128


def _pack32(h):
    h4 = h.reshape(-1, 4, DH)
    return jnp.concatenate([h4[:, g, :] for g in range(4)], axis=1)


def _tc_first(x, W1, degp):
    def body(x_ref, w_ref, d_ref, o_ref):
        dis = lax.rsqrt(d_ref[0] + d_ref[1] + 1.0)
        h = jnp.dot(x_ref[...], w_ref[...], preferred_element_type=jnp.float32)
        o_ref[...] = dis * _pack32(h)

    return pl.pallas_call(
        body,
        grid=(_G,),
        in_specs=[
            pl.BlockSpec((_R, D_IN), lambda i: (i, 0)),
            pl.BlockSpec((D_IN, DH), lambda i: (0, 0)),
            pl.BlockSpec((2, _RH, 128), lambda i: (0, i, 0)),
        ],
        out_specs=pl.BlockSpec((_RH, 128), lambda i: (i, 0)),
        out_shape=jax.ShapeDtypeStruct((NPAD * DH // 128, 128), jnp.float32),
    )(x, W1, degp)


def _tc_mid(degp, sp, hp, W2blk, b1t):
    def body(d_ref, sp_ref, hp_ref, w_ref, b_ref, o_ref):
        dis = lax.rsqrt(d_ref[0] + d_ref[1] + 1.0)
        seg = sp_ref[0] + sp_ref[1] + hp_ref[...]
        h1 = jnp.maximum(dis * seg + b_ref[...], 0.0)
        h2 = jnp.dot(h1, w_ref[...], preferred_element_type=jnp.float32)
        o_ref[...] = dis * h2

    return pl.pallas_call(
        body,
        grid=(_G,),
        in_specs=[
            pl.BlockSpec((2, _RH, 128), lambda i: (0, i, 0)),
            pl.BlockSpec((2, _RH, 128), lambda i: (0, i, 0)),
            pl.BlockSpec((_RH, 128), lambda i: (i, 0)),
            pl.BlockSpec((128, 128), lambda i: (0, 0)),
            pl.BlockSpec((1, 128), lambda i: (0, 0)),
        ],
        out_specs=pl.BlockSpec((_RH, 128), lambda i: (i, 0)),
        out_shape=jax.ShapeDtypeStruct((NPAD * DH // 128, 128), jnp.float32),
    )(degp, sp, hp, W2blk, b1t)


def _tc_last(degp, sp, hp, b2t, Wlblk, blt):
    def body(d_ref, sp_ref, hp_ref, b2_ref, wl_ref, bl_ref, o_ref):
        dis = lax.rsqrt(d_ref[0] + d_ref[1] + 1.0)
        seg = sp_ref[0] + sp_ref[1] + hp_ref[...]
        h2 = jnp.maximum(dis * seg + b2_ref[...], 0.0)
        o = jnp.dot(h2, wl_ref[...], preferred_element_type=jnp.float32) + bl_ref[...]
        parts = []
        for g in range(4):
            og = o[:, DO * g:DO * (g + 1)]
            m = jnp.max(og, axis=1, keepdims=True)
            lse = jnp.log(jnp.sum(jnp.exp(og - m), axis=1, keepdims=True)) + m
            parts.append((og - lse).reshape(-1, 1, DO))
        o_ref[...] = jnp.concatenate(parts, axis=1).reshape(_R, DO)

    return pl.pallas_call(
        body,
        grid=(_G,),
        in_specs=[
            pl.BlockSpec((2, _RH, 128), lambda i: (0, i, 0)),
            pl.BlockSpec((2, _RH, 128), lambda i: (0, i, 0)),
            pl.BlockSpec((_RH, 128), lambda i: (i, 0)),
            pl.BlockSpec((1, 128), lambda i: (0, 0)),
            pl.BlockSpec((128, 4 * DO), lambda i: (0, 0)),
            pl.BlockSpec((1, 4 * DO), lambda i: (0, 0)),
        ],
        out_specs=pl.BlockSpec((_R, DO), lambda i: (i, 0)),
        out_shape=jax.ShapeDtypeStruct((N, DO), jnp.float32),
    )(degp, sp, hp, b2t, Wlblk, blt)


def _blockdiag4(W):
    k, m = W.shape
    out = jnp.zeros((4 * k, 4 * m), W.dtype)
    for g in range(4):
        out = out.at[g * k:(g + 1) * k, g * m:(g + 1) * m].set(W)
    return out


def kernel(x, edge_index, W1, b1, W2, b2, Wl, bl):
    ei = edge_index.astype(jnp.int32)
    src_r = ei[0].reshape(NW, CH, CW)
    dst_r = ei[1].reshape(NW, CH, CW)
    W2blk = _blockdiag4(W2)
    Wlblk = _blockdiag4(Wl)
    b1t = jnp.tile(b1.reshape(1, DH), (1, 4))
    b2t = jnp.tile(b2.reshape(1, DH), (1, 4))
    blt = jnp.tile(bl.reshape(1, DO), (1, 4))

    degp = _sc_degree(dst_r).reshape(2, NPAD * DH // 128, 128)
    h1p = _tc_first(x, W1, degp)
    s1p = _sc_propagate(h1p.reshape(NPAD, DH), src_r, dst_r)
    h2p = _tc_mid(degp, s1p.reshape(2, NPAD * DH // 128, 128), h1p, W2blk, b1t)
    s2p = _sc_propagate(h2p.reshape(NPAD, DH), src_r, dst_r)
    return _tc_last(degp, s2p.reshape(2, NPAD * DH // 128, 128), h2p, b2t, Wlblk, blt)


# speedup vs baseline: 1.4082x; 1.0998x over previous
import functools

import jax
import jax.numpy as jnp
from jax import lax
from jax.experimental import pallas as pl
from jax.experimental.pallas import tpu as pltpu
from jax.experimental.pallas import tpu_sc as plsc

N = 10000
D_IN = 128
DH = 32
DO = 16
NC = 2
NS = 16
NW = NC * NS
EPW = 320000 // NW
CW = 125
CH = EPW // CW
NPAD = 10240
RPT = NPAD // NS
ZB = 128
NB = 8


def _mesh():
    return plsc.VectorSubcoreMesh(
        core_axis_name="c", subcore_axis_name="s", num_cores=NC, num_subcores=NS
    )


_SC_PARAMS = pltpu.CompilerParams(use_tc_tiling_on_sc=False)


def _sc_degree(ei4):
    @functools.partial(
        pl.kernel,
        out_type=jax.ShapeDtypeStruct((NC, NPAD, DH), jnp.float32),
        mesh=_mesh(),
        compiler_params=_SC_PARAMS,
        scratch_types=[
            pltpu.VMEM((CH, CW), jnp.int32),
            pltpu.VMEM((ZB, 16), jnp.float32),
            pltpu.VMEM((CW, 16), jnp.float32),
            pltpu.VMEM((RPT, 16), jnp.float32),
            pltpu.VMEM((RPT, DH), jnp.float32),
            pltpu.VMEM_SHARED((NPAD, 16), jnp.float32),
            pltpu.SemaphoreType.DMA,
        ],
    )
    def k(ei_hbm, out_hbm, idx_v, zbuf_v, ones_v, tmp_v, dup_v, acc_sh, sem):
        c = lax.axis_index("c")
        s = lax.axis_index("s")
        wid = c * NS + s
        pltpu.sync_copy(ei_hbm.at[1, wid], idx_v)

        def zrow(i, carry):
            zbuf_v[i, :] = jnp.zeros((16,), jnp.float32)
            return carry

        lax.fori_loop(0, ZB, zrow, 0)

        def orow(i, carry):
            ones_v[i, :] = jnp.full((16,), 1.0, jnp.float32)
            return carry

        lax.fori_loop(0, CW, orow, 0)
        base = s * RPT
        for r in range(RPT // ZB):
            pltpu.sync_copy(zbuf_v, acc_sh.at[pl.ds(base + r * ZB, ZB)])
        plsc.subcore_barrier()

        def chunk(j, carry):
            pltpu.async_copy(ones_v, acc_sh.at[idx_v.at[j]], sem, add=True)
            return carry

        lax.fori_loop(0, CH, chunk, 0)

        def drain(j, carry):
            pltpu.make_async_copy(ones_v, acc_sh.at[idx_v.at[j]], sem).wait()
            return carry

        lax.fori_loop(0, CH, drain, 0)
        plsc.subcore_barrier()
        pltpu.sync_copy(acc_sh.at[pl.ds(base, RPT)], tmp_v)

        def dup(i, carry):
            row = tmp_v[i, :]
            dup_v[i, pl.ds(0, 16)] = row
            dup_v[i, pl.ds(16, 16)] = row
            return carry

        lax.fori_loop(0, RPT, dup, 0)
        pltpu.sync_copy(dup_v, out_hbm.at[c, pl.ds(base, RPT)])

    return k(ei4)


def _sc_propagate(hp, ei4):
    @functools.partial(
        pl.kernel,
        out_type=jax.ShapeDtypeStruct((NC, NPAD, DH), jnp.float32),
        mesh=_mesh(),
        compiler_params=_SC_PARAMS,
        scratch_types=[
            pltpu.VMEM((CH, CW), jnp.int32),
            pltpu.VMEM((CH, CW), jnp.int32),
            pltpu.VMEM((ZB, DH), jnp.float32),
            pltpu.VMEM((NB, CW, DH), jnp.float32),
            pltpu.VMEM_SHARED((NPAD, DH), jnp.float32),
            pltpu.SemaphoreType.DMA((NB,)),
            pltpu.SemaphoreType.DMA((NB,)),
        ],
    )
    def k(hp_hbm, ei_hbm, out_hbm, sidx_v, didx_v, zbuf_v, rows_v,
          acc_sh, gsem, ssem):
        c = lax.axis_index("c")
        s = lax.axis_index("s")
        wid = c * NS + s
        pltpu.sync_copy(ei_hbm.at[0, wid], sidx_v)
        pltpu.sync_copy(ei_hbm.at[1, wid], didx_v)

        def zrow(i, carry):
            zbuf_v[i, pl.ds(0, 16)] = jnp.zeros((16,), jnp.float32)
            zbuf_v[i, pl.ds(16, 16)] = jnp.zeros((16,), jnp.float32)
            return carry

        lax.fori_loop(0, ZB, zrow, 0)
        base = s * RPT
        for r in range(RPT // ZB):
            pltpu.sync_copy(zbuf_v, acc_sh.at[pl.ds(base + r * ZB, ZB)])
        plsc.subcore_barrier()

        def gather(i, b):
            pltpu.async_copy(hp_hbm.at[sidx_v.at[i]], rows_v.at[b], gsem.at[b])

        def gather_wait(i, b):
            pltpu.make_async_copy(
                hp_hbm.at[sidx_v.at[i]], rows_v.at[b], gsem.at[b]
            ).wait()

        def scatter(i, b):
            pltpu.async_copy(
                rows_v.at[b], acc_sh.at[didx_v.at[i]], ssem.at[b], add=True
            )

        def scatter_wait(i, b):
            pltpu.make_async_copy(
                rows_v.at[b], acc_sh.at[didx_v.at[i]], ssem.at[b]
            ).wait()

        def step(i, p, wait_prev_scatter, prefetch):
            gather_wait(i, p)
            scatter(i, p)
            if prefetch:
                bp = (p + NB - 1) % NB
                if wait_prev_scatter:
                    scatter_wait(i - 1, bp)
                gather(i + NB - 1, bp)

        for b in range(NB - 1):
            gather(b, b)
        step(0, 0, False, True)
        for p in range(1, NB):
            step(p, p, True, True)

        def group(g, carry):
            for p in range(NB):
                step(g * NB + p, p, True, True)
            return carry

        lax.fori_loop(1, CH // NB - 1, group, 0)
        i0 = CH - NB
        step(i0, 0, True, True)
        for p in range(1, NB):
            step(i0 + p, p, False, False)
        for b in range(NB):
            scatter_wait(i0 + b, b)
        plsc.subcore_barrier()
        pltpu.sync_copy(acc_sh.at[pl.ds(base, RPT)], out_hbm.at[c, pl.ds(base, RPT)])

    return k(hp, ei4)


_R = 2560
_G = NPAD // _R
_RD = _R * 16 // 128
_RH = _R * DH // 128


def _pack32(h):
    h4 = h.reshape(-1, 4, DH)
    return jnp.concatenate([h4[:, g, :] for g in range(4)], axis=1)


def _tc_first(x, W1, degp):
    def body(x_ref, w_ref, d_ref, o_ref):
        dis = lax.rsqrt(d_ref[0] + d_ref[1] + 1.0)
        h = jnp.dot(x_ref[...], w_ref[...], preferred_element_type=jnp.float32)
        o_ref[...] = dis * _pack32(h)

    return pl.pallas_call(
        body,
        grid=(_G,),
        in_specs=[
            pl.BlockSpec((_R, D_IN), lambda i: (i, 0)),
            pl.BlockSpec((D_IN, DH), lambda i: (0, 0)),
            pl.BlockSpec((2, _RH, 128), lambda i: (0, i, 0)),
        ],
        out_specs=pl.BlockSpec((_RH, 128), lambda i: (i, 0)),
        out_shape=jax.ShapeDtypeStruct((NPAD * DH // 128, 128), jnp.float32),
    )(x, W1, degp)


def _tc_mid(degp, sp, hp, W2blk, b1t):
    def body(d_ref, sp_ref, hp_ref, w_ref, b_ref, o_ref):
        dis = lax.rsqrt(d_ref[0] + d_ref[1] + 1.0)
        seg = sp_ref[0] + sp_ref[1] + hp_ref[...]
        h1 = jnp.maximum(dis * seg + b_ref[...], 0.0)
        h2 = jnp.dot(h1, w_ref[...], preferred_element_type=jnp.float32)
        o_ref[...] = dis * h2

    return pl.pallas_call(
        body,
        grid=(_G,),
        in_specs=[
            pl.BlockSpec((2, _RH, 128), lambda i: (0, i, 0)),
            pl.BlockSpec((2, _RH, 128), lambda i: (0, i, 0)),
            pl.BlockSpec((_RH, 128), lambda i: (i, 0)),
            pl.BlockSpec((128, 128), lambda i: (0, 0)),
            pl.BlockSpec((1, 128), lambda i: (0, 0)),
        ],
        out_specs=pl.BlockSpec((_RH, 128), lambda i: (i, 0)),
        out_shape=jax.ShapeDtypeStruct((NPAD * DH // 128, 128), jnp.float32),
    )(degp, sp, hp, W2blk, b1t)


def _tc_last(degp, sp, hp, b2t, Wlblk, blt):
    def body(d_ref, sp_ref, hp_ref, b2_ref, wl_ref, bl_ref, o_ref):
        dis = lax.rsqrt(d_ref[0] + d_ref[1] + 1.0)
        seg = sp_ref[0] + sp_ref[1] + hp_ref[...]
        h2 = jnp.maximum(dis * seg + b2_ref[...], 0.0)
        o = jnp.dot(h2, wl_ref[...], preferred_element_type=jnp.float32) + bl_ref[...]
        parts = []
        for g in range(4):
            og = o[:, DO * g:DO * (g + 1)]
            m = jnp.max(og, axis=1, keepdims=True)
            lse = jnp.log(jnp.sum(jnp.exp(og - m), axis=1, keepdims=True)) + m
            parts.append((og - lse).reshape(-1, 1, DO))
        o_ref[...] = jnp.concatenate(parts, axis=1).reshape(_R, DO)

    return pl.pallas_call(
        body,
        grid=(_G,),
        in_specs=[
            pl.BlockSpec((2, _RH, 128), lambda i: (0, i, 0)),
            pl.BlockSpec((2, _RH, 128), lambda i: (0, i, 0)),
            pl.BlockSpec((_RH, 128), lambda i: (i, 0)),
            pl.BlockSpec((1, 128), lambda i: (0, 0)),
            pl.BlockSpec((128, 4 * DO), lambda i: (0, 0)),
            pl.BlockSpec((1, 4 * DO), lambda i: (0, 0)),
        ],
        out_specs=pl.BlockSpec((_R, DO), lambda i: (i, 0)),
        out_shape=jax.ShapeDtypeStruct((N, DO), jnp.float32),
    )(degp, sp, hp, b2t, Wlblk, blt)


def _blockdiag4(W):
    k, m = W.shape
    out = jnp.zeros((4 * k, 4 * m), W.dtype)
    for g in range(4):
        out = out.at[g * k:(g + 1) * k, g * m:(g + 1) * m].set(W)
    return out


def kernel(x, edge_index, W1, b1, W2, b2, Wl, bl):
    ei4 = edge_index.astype(jnp.int32).reshape(2, NW, CH, CW)
    W2blk = _blockdiag4(W2)
    Wlblk = _blockdiag4(Wl)
    b1t = jnp.tile(b1.reshape(1, DH), (1, 4))
    b2t = jnp.tile(b2.reshape(1, DH), (1, 4))
    blt = jnp.tile(bl.reshape(1, DO), (1, 4))

    degp = _sc_degree(ei4).reshape(2, NPAD * DH // 128, 128)
    h1p = _tc_first(x, W1, degp)
    s1p = _sc_propagate(h1p.reshape(NPAD, DH), ei4)
    h2p = _tc_mid(degp, s1p.reshape(2, NPAD * DH // 128, 128), h1p, W2blk, b1t)
    s2p = _sc_propagate(h2p.reshape(NPAD, DH), ei4)
    return _tc_last(degp, s2p.reshape(2, NPAD * DH // 128, 128), h2p, b2t, Wlblk, blt)


# speedup vs baseline: 1.4773x; 1.0491x over previous
import functools

import jax
import jax.numpy as jnp
from jax import lax
from jax.experimental import pallas as pl
from jax.experimental.pallas import tpu as pltpu
from jax.experimental.pallas import tpu_sc as plsc

N = 10000
D_IN = 128
DH = 32
DO = 16
NC = 2
NS = 16
NW = NC * NS
EPW = 320000 // NW
CW = 125
CH = EPW // CW
NPAD = 10240
RPT = NPAD // NS
ZB = 128
NB = 8


def _mesh():
    return plsc.VectorSubcoreMesh(
        core_axis_name="c", subcore_axis_name="s", num_cores=NC, num_subcores=NS
    )


_SC_PARAMS = pltpu.CompilerParams(use_tc_tiling_on_sc=False)


def _sc_degree(ei4):
    @functools.partial(
        pl.kernel,
        out_type=jax.ShapeDtypeStruct((NC, NPAD, DH), jnp.float32),
        mesh=_mesh(),
        compiler_params=_SC_PARAMS,
        scratch_types=[
            pltpu.VMEM((CH, CW), jnp.int32),
            pltpu.VMEM((ZB, 16), jnp.float32),
            pltpu.VMEM((CW, 16), jnp.float32),
            pltpu.VMEM((RPT, 16), jnp.float32),
            pltpu.VMEM((RPT, DH), jnp.float32),
            pltpu.VMEM_SHARED((NPAD, 16), jnp.float32),
            pltpu.SemaphoreType.DMA,
        ],
    )
    def k(ei_hbm, out_hbm, idx_v, zbuf_v, ones_v, tmp_v, dup_v, acc_sh, sem):
        c = lax.axis_index("c")
        s = lax.axis_index("s")
        wid = c * NS + s
        pltpu.sync_copy(ei_hbm.at[1, wid], idx_v)

        def zrow(i, carry):
            zbuf_v[i, :] = jnp.zeros((16,), jnp.float32)
            return carry

        lax.fori_loop(0, ZB, zrow, 0)

        def orow(i, carry):
            ones_v[i, :] = jnp.full((16,), 1.0, jnp.float32)
            return carry

        lax.fori_loop(0, CW, orow, 0)
        base = s * RPT
        for r in range(RPT // ZB):
            pltpu.sync_copy(zbuf_v, acc_sh.at[pl.ds(base + r * ZB, ZB)])
        plsc.subcore_barrier()

        def chunk(j, carry):
            pltpu.async_copy(ones_v, acc_sh.at[idx_v.at[j]], sem, add=True)
            return carry

        lax.fori_loop(0, CH, chunk, 0)

        def drain(j, carry):
            pltpu.make_async_copy(ones_v, acc_sh.at[idx_v.at[j]], sem).wait()
            return carry

        lax.fori_loop(0, CH, drain, 0)
        plsc.subcore_barrier()
        pltpu.sync_copy(acc_sh.at[pl.ds(base, RPT)], tmp_v)

        def dup(i, carry):
            row = tmp_v[i, :]
            dup_v[i, pl.ds(0, 16)] = row
            dup_v[i, pl.ds(16, 16)] = row
            return carry

        lax.fori_loop(0, RPT, dup, 0)
        pltpu.sync_copy(dup_v, out_hbm.at[c, pl.ds(base, RPT)])

    return k(ei4)


def _sc_propagate(hp, ei4):
    @functools.partial(
        pl.kernel,
        out_type=jax.ShapeDtypeStruct((NC, NPAD, DH), jnp.float32),
        mesh=_mesh(),
        compiler_params=_SC_PARAMS,
        scratch_types=[
            pltpu.VMEM((CH, CW), jnp.int32),
            pltpu.VMEM((CH, CW), jnp.int32),
            pltpu.VMEM((ZB, DH), jnp.float32),
            pltpu.VMEM((NB, CW, DH), jnp.float32),
            pltpu.VMEM_SHARED((NPAD, DH), jnp.float32),
            pltpu.SemaphoreType.DMA((NB,)),
            pltpu.SemaphoreType.DMA((NB,)),
        ],
    )
    def k(hp_hbm, ei_hbm, out_hbm, sidx_v, didx_v, zbuf_v, rows_v,
          acc_sh, gsem, ssem):
        c = lax.axis_index("c")
        s = lax.axis_index("s")
        wid = c * NS + s
        pltpu.sync_copy(ei_hbm.at[0, wid], sidx_v)
        pltpu.sync_copy(ei_hbm.at[1, wid], didx_v)

        def zrow(i, carry):
            zbuf_v[i, pl.ds(0, 16)] = jnp.zeros((16,), jnp.float32)
            zbuf_v[i, pl.ds(16, 16)] = jnp.zeros((16,), jnp.float32)
            return carry

        lax.fori_loop(0, ZB, zrow, 0)
        base = s * RPT
        for r in range(RPT // ZB):
            pltpu.sync_copy(zbuf_v, acc_sh.at[pl.ds(base + r * ZB, ZB)])
        plsc.subcore_barrier()

        def gather(i, b):
            pltpu.async_copy(hp_hbm.at[sidx_v.at[i]], rows_v.at[b], gsem.at[b])

        def gather_wait(i, b):
            pltpu.make_async_copy(
                hp_hbm.at[sidx_v.at[i]], rows_v.at[b], gsem.at[b]
            ).wait()

        def scatter(i, b):
            pltpu.async_copy(
                rows_v.at[b], acc_sh.at[didx_v.at[i]], ssem.at[b], add=True
            )

        def scatter_wait(i, b):
            pltpu.make_async_copy(
                rows_v.at[b], acc_sh.at[didx_v.at[i]], ssem.at[b]
            ).wait()

        def step(i, p, wait_prev_scatter, prefetch):
            gather_wait(i, p)
            scatter(i, p)
            if prefetch:
                bp = (p + NB - 1) % NB
                if wait_prev_scatter:
                    scatter_wait(i - 1, bp)
                gather(i + NB - 1, bp)

        for b in range(NB - 1):
            gather(b, b)
        step(0, 0, False, True)
        for p in range(1, NB):
            step(p, p, True, True)

        def group(g, carry):
            for p in range(NB):
                step(g * NB + p, p, True, True)
            return carry

        lax.fori_loop(1, CH // NB - 1, group, 0)
        i0 = CH - NB
        step(i0, 0, True, True)
        for p in range(1, NB):
            step(i0 + p, p, False, False)
        for b in range(NB):
            scatter_wait(i0 + b, b)
        plsc.subcore_barrier()
        pltpu.sync_copy(acc_sh.at[pl.ds(base, RPT)], out_hbm.at[c, pl.ds(base, RPT)])

    return k(hp, ei4)


_R = 2560
_G = NPAD // _R
_RD = _R * 16 // 128
_RH = _R * DH // 128


def _pack32(h):
    h4 = h.reshape(-1, 4, DH)
    return jnp.concatenate([h4[:, g, :] for g in range(4)], axis=1)


def _tc_mm(x, W1):
    def body(x_ref, w_ref, o_ref):
        h = jnp.dot(x_ref[...], w_ref[...], preferred_element_type=jnp.float32)
        o_ref[...] = _pack32(h)

    return pl.pallas_call(
        body,
        grid=(_G,),
        in_specs=[
            pl.BlockSpec((_R, D_IN), lambda i: (i, 0)),
            pl.BlockSpec((D_IN, DH), lambda i: (0, 0)),
        ],
        out_specs=pl.BlockSpec((_RH, 128), lambda i: (i, 0)),
        out_shape=jax.ShapeDtypeStruct((NPAD * DH // 128, 128), jnp.float32),
    )(x, W1)


def _tc_scale(degp, h):
    def body(d_ref, h_ref, o_ref):
        dis = lax.rsqrt(d_ref[0] + d_ref[1] + 1.0)
        o_ref[...] = dis * h_ref[...]

    return pl.pallas_call(
        body,
        grid=(_G,),
        in_specs=[
            pl.BlockSpec((2, _RH, 128), lambda i: (0, i, 0)),
            pl.BlockSpec((_RH, 128), lambda i: (i, 0)),
        ],
        out_specs=pl.BlockSpec((_RH, 128), lambda i: (i, 0)),
        out_shape=jax.ShapeDtypeStruct((NPAD * DH // 128, 128), jnp.float32),
    )(degp, h)


def _tc_mid(degp, sp, hp, W2blk, b1t):
    def body(d_ref, sp_ref, hp_ref, w_ref, b_ref, o_ref):
        dis = lax.rsqrt(d_ref[0] + d_ref[1] + 1.0)
        seg = sp_ref[0] + sp_ref[1] + hp_ref[...]
        h1 = jnp.maximum(dis * seg + b_ref[...], 0.0)
        h2 = jnp.dot(h1, w_ref[...], preferred_element_type=jnp.float32)
        o_ref[...] = dis * h2

    return pl.pallas_call(
        body,
        grid=(_G,),
        in_specs=[
            pl.BlockSpec((2, _RH, 128), lambda i: (0, i, 0)),
            pl.BlockSpec((2, _RH, 128), lambda i: (0, i, 0)),
            pl.BlockSpec((_RH, 128), lambda i: (i, 0)),
            pl.BlockSpec((128, 128), lambda i: (0, 0)),
            pl.BlockSpec((1, 128), lambda i: (0, 0)),
        ],
        out_specs=pl.BlockSpec((_RH, 128), lambda i: (i, 0)),
        out_shape=jax.ShapeDtypeStruct((NPAD * DH // 128, 128), jnp.float32),
    )(degp, sp, hp, W2blk, b1t)


def _tc_last(degp, sp, hp, b2t, Wlblk, blt, Gblk):
    def body(d_ref, sp_ref, hp_ref, b2_ref, wl_ref, bl_ref, g_ref, o_ref):
        dis = lax.rsqrt(d_ref[0] + d_ref[1] + 1.0)
        seg = sp_ref[0] + sp_ref[1] + hp_ref[...]
        h2 = jnp.maximum(dis * seg + b2_ref[...], 0.0)
        o = jnp.dot(h2, wl_ref[...], preferred_element_type=jnp.float32) + bl_ref[...]
        m = jnp.max(o, axis=1, keepdims=True)
        e = jnp.exp(o - m)
        ssum = jnp.dot(e, g_ref[...], preferred_element_type=jnp.float32)
        op = o - m - jnp.log(ssum)
        parts = [op[:, DO * g:DO * (g + 1)].reshape(-1, 1, DO) for g in range(4)]
        o_ref[...] = jnp.concatenate(parts, axis=1).reshape(_R, DO)

    return pl.pallas_call(
        body,
        grid=(_G,),
        in_specs=[
            pl.BlockSpec((2, _RH, 128), lambda i: (0, i, 0)),
            pl.BlockSpec((2, _RH, 128), lambda i: (0, i, 0)),
            pl.BlockSpec((_RH, 128), lambda i: (i, 0)),
            pl.BlockSpec((1, 128), lambda i: (0, 0)),
            pl.BlockSpec((128, 4 * DO), lambda i: (0, 0)),
            pl.BlockSpec((1, 4 * DO), lambda i: (0, 0)),
            pl.BlockSpec((4 * DO, 4 * DO), lambda i: (0, 0)),
        ],
        out_specs=pl.BlockSpec((_R, DO), lambda i: (i, 0)),
        out_shape=jax.ShapeDtypeStruct((N, DO), jnp.float32),
    )(degp, sp, hp, b2t, Wlblk, blt, Gblk)


def _blockdiag4(W):
    k, m = W.shape
    out = jnp.zeros((4 * k, 4 * m), W.dtype)
    for g in range(4):
        out = out.at[g * k:(g + 1) * k, g * m:(g + 1) * m].set(W)
    return out


def kernel(x, edge_index, W1, b1, W2, b2, Wl, bl):
    ei4 = edge_index.astype(jnp.int32).reshape(2, NW, CH, CW)
    W2blk = _blockdiag4(W2)
    Wlblk = _blockdiag4(Wl)
    b1t = jnp.tile(b1.reshape(1, DH), (1, 4))
    b2t = jnp.tile(b2.reshape(1, DH), (1, 4))
    blt = jnp.tile(bl.reshape(1, DO), (1, 4))

    Gblk = _blockdiag4(jnp.ones((DO, DO), jnp.float32))

    degp = _sc_degree(ei4).reshape(2, NPAD * DH // 128, 128)
    h1 = _tc_mm(x, W1)
    h1p = _tc_scale(degp, h1)
    s1p = _sc_propagate(h1p.reshape(NPAD, DH), ei4)
    h2p = _tc_mid(degp, s1p.reshape(2, NPAD * DH // 128, 128), h1p, W2blk, b1t)
    s2p = _sc_propagate(h2p.reshape(NPAD, DH), ei4)
    return _tc_last(degp, s2p.reshape(2, NPAD * DH // 128, 128), h2p, b2t, Wlblk, blt, Gblk)
